# baseline (speedup 1.0000x reference)
import functools

import jax
import jax.numpy as jnp
from jax.experimental import pallas as pl
from jax.experimental.pallas import tpu as pltpu

_C = 128
_IMGS = 4


def _silu_of_2h(h):
    return h + h * jnp.tanh(h)


def _silu_of_2h_bf16(h, b16=None):
    h16 = h.astype(jnp.bfloat16)
    if b16 is not None:
        h16 = h16 + b16
    t16 = jnp.tanh(h16)
    return h16 + h16 * t16


def _shift_pair(v, w):
    c, hw = v.shape
    z = jnp.zeros((c, w), v.dtype)
    up = jnp.concatenate([z, v[:, :hw - w]], axis=1)
    dn = jnp.concatenate([v[:, w:], z], axis=1)
    return up, dn


def _pipe_kernel(x_ref, p_ref, o_ref, *, imgs, w):
    p = p_ref[...] * 0.5
    p16 = p.astype(jnp.bfloat16)
    w1 = p16[:, 0:385]
    w2 = p16[:, 385:513]
    b216 = p16[:, 513:514]
    w4 = p16[:, 518:646]
    b416 = p16[:, 646:647]
    w5 = p16[:, 647:1032]
    w3a16 = p16[:, 514:515]
    w3b16 = p16[:, 515:516]
    w3c16 = p16[:, 516:517]
    b316 = p16[:, 517:518]
    hw = o_ref.shape[-1]
    ones = jnp.ones((1, hw), jnp.bfloat16)

    for i in range(imgs):
        x = x_ref[i]
        x16 = x.astype(jnp.bfloat16)

        up, dn = _shift_pair(x16, w)
        xcat = jnp.concatenate([up, x16, dn, ones], axis=0)
        h = jnp.dot(w1, xcat, preferred_element_type=jnp.float32)
        y16 = _silu_of_2h_bf16(h)

        h = jnp.dot(w2, y16, preferred_element_type=jnp.float32)
        y16 = _silu_of_2h_bf16(h, b216)

        up, dn = _shift_pair(y16, w)
        h = up * w3a16 + y16 * w3b16 + dn * w3c16 + b316
        y16 = _silu_of_2h_bf16(h)

        h = jnp.dot(w4, y16, preferred_element_type=jnp.float32)
        y16 = _silu_of_2h_bf16(h, b416)

        up, dn = _shift_pair(y16, w)
        ycat = jnp.concatenate([up, y16, dn, ones], axis=0)
        h = jnp.dot(w5, ycat, preferred_element_type=jnp.float32)
        y16 = _silu_of_2h_bf16(h)

        o_ref[i] = x + y16


def kernel(x_nchw, pslab):
    N, C, H, W = x_nchw.shape
    HW = H * W
    xk = x_nchw.reshape(N, C, HW)

    out = pl.pallas_call(
        functools.partial(_pipe_kernel, imgs=_IMGS, w=W),
        out_shape=jax.ShapeDtypeStruct((N, C, HW), x_nchw.dtype),
        grid=(N // _IMGS,),
        in_specs=[
            pl.BlockSpec((_IMGS, C, HW), lambda i: (i, 0, 0)),
            pl.BlockSpec(pslab.shape, lambda i: (0, 0)),
        ],
        out_specs=pl.BlockSpec((_IMGS, C, HW), lambda i: (i, 0, 0)),
        compiler_params=pltpu.CompilerParams(
            dimension_semantics=("arbitrary",),
            vmem_limit_bytes=64 * 1024 * 1024),
    )(xk, pslab)

    return out.reshape(N, C, H, W)


# speedup vs baseline: 1.0123x; 1.0123x over previous
import functools

import jax
import jax.numpy as jnp
from jax.experimental import pallas as pl
from jax.experimental.pallas import tpu as pltpu

_C = 128
_IMGS = 8


def _silu_of_2h(h):
    return h + h * jnp.tanh(h)


def _silu_of_2h_bf16(h, b16=None):
    h16 = h.astype(jnp.bfloat16)
    if b16 is not None:
        h16 = h16 + b16
    t16 = jnp.tanh(h16)
    return h16 + h16 * t16


def _shift_pair(v, w):
    c, hw = v.shape
    z = jnp.zeros((c, w), v.dtype)
    up = jnp.concatenate([z, v[:, :hw - w]], axis=1)
    dn = jnp.concatenate([v[:, w:], z], axis=1)
    return up, dn


def _pipe_kernel(x_ref, p_ref, o_ref, *, imgs, w):
    p = p_ref[...] * 0.5
    p16 = p.astype(jnp.bfloat16)
    w1 = p16[:, 0:385]
    w2 = p16[:, 385:513]
    b216 = p16[:, 513:514]
    w4 = p16[:, 518:646]
    b416 = p16[:, 646:647]
    w5 = p16[:, 647:1032]
    w3a16 = p16[:, 514:515]
    w3b16 = p16[:, 515:516]
    w3c16 = p16[:, 516:517]
    b316 = p16[:, 517:518]
    hw = o_ref.shape[-1]
    ones = jnp.ones((1, hw), jnp.bfloat16)

    for i in range(imgs):
        x = x_ref[i]
        x16 = x.astype(jnp.bfloat16)

        up, dn = _shift_pair(x16, w)
        xcat = jnp.concatenate([up, x16, dn, ones], axis=0)
        h = jnp.dot(w1, xcat, preferred_element_type=jnp.float32)
        y16 = _silu_of_2h_bf16(h)

        h = jnp.dot(w2, y16, preferred_element_type=jnp.float32)
        y16 = _silu_of_2h_bf16(h, b216)

        up, dn = _shift_pair(y16, w)
        h = up * w3a16 + y16 * w3b16 + dn * w3c16 + b316
        y16 = _silu_of_2h_bf16(h)

        h = jnp.dot(w4, y16, preferred_element_type=jnp.float32)
        y16 = _silu_of_2h_bf16(h, b416)

        up, dn = _shift_pair(y16, w)
        ycat = jnp.concatenate([up, y16, dn, ones], axis=0)
        h = jnp.dot(w5, ycat, preferred_element_type=jnp.float32)
        y16 = _silu_of_2h_bf16(h)

        o_ref[i] = x + y16


def kernel(x_nchw, pslab):
    N, C, H, W = x_nchw.shape
    HW = H * W
    xk = x_nchw.reshape(N, C, HW)

    out = pl.pallas_call(
        functools.partial(_pipe_kernel, imgs=_IMGS, w=W),
        out_shape=jax.ShapeDtypeStruct((N, C, HW), x_nchw.dtype),
        grid=(N // _IMGS,),
        in_specs=[
            pl.BlockSpec((_IMGS, C, HW), lambda i: (i, 0, 0)),
            pl.BlockSpec(pslab.shape, lambda i: (0, 0)),
        ],
        out_specs=pl.BlockSpec((_IMGS, C, HW), lambda i: (i, 0, 0)),
        compiler_params=pltpu.CompilerParams(
            dimension_semantics=("arbitrary",),
            vmem_limit_bytes=64 * 1024 * 1024),
    )(xk, pslab)

    return out.reshape(N, C, H, W)
